# baseline (device time: 13178 ns/iter reference)
import jax
import jax.numpy as jnp
from jax import lax
from jax.experimental import pallas as pl
from jax.experimental.pallas import tpu as pltpu


def kernel(partial, gamma):
    _, m_total, d = partial.shape
    m_half = m_total // 2

    def body(p_ref, g_ref, out_ref, send_buf, recv_buf, send_sem, recv_sem):
        my_x = lax.axis_index("x")
        my_y = lax.axis_index("y")
        my_z = lax.axis_index("z")
        peer = (1 - my_x, my_y, my_z)

        send_buf[...] = p_ref[0, pl.ds((1 - my_x) * m_half, m_half), :].astype(
            jnp.bfloat16
        )

        barrier_sem = pltpu.get_barrier_semaphore()
        pl.semaphore_signal(
            barrier_sem,
            inc=1,
            device_id=peer,
            device_id_type=pl.DeviceIdType.MESH,
        )
        pl.semaphore_wait(barrier_sem, 1)

        rdma = pltpu.make_async_remote_copy(
            src_ref=send_buf,
            dst_ref=recv_buf,
            send_sem=send_sem,
            recv_sem=recv_sem,
            device_id=peer,
            device_id_type=pl.DeviceIdType.MESH,
        )
        rdma.start()
        local = p_ref[0, pl.ds(my_x * m_half, m_half), :]
        rdma.wait()

        y = local + recv_buf[...].astype(jnp.float32)
        rms = jnp.sqrt(jnp.mean(y * y, axis=-1, keepdims=True) + 1e-6)
        out_ref[...] = y / rms * g_ref[...]

    gamma2 = gamma.reshape(1, d)
    return pl.pallas_call(
        body,
        out_shape=jax.ShapeDtypeStruct((m_half, d), jnp.float32),
        in_specs=[
            pl.BlockSpec(memory_space=pltpu.VMEM),
            pl.BlockSpec(memory_space=pltpu.VMEM),
        ],
        out_specs=pl.BlockSpec(memory_space=pltpu.VMEM),
        scratch_shapes=[
            pltpu.VMEM((m_half, d), jnp.bfloat16),
            pltpu.VMEM((m_half, d), jnp.bfloat16),
            pltpu.SemaphoreType.DMA,
            pltpu.SemaphoreType.DMA,
        ],
        compiler_params=pltpu.CompilerParams(collective_id=0),
    )(partial, gamma2)


# device time: 13034 ns/iter; 1.0110x vs baseline; 1.0110x over previous
import jax
import jax.numpy as jnp
from jax import lax
from jax.experimental import pallas as pl
from jax.experimental.pallas import tpu as pltpu


NC = 4


def kernel(partial, gamma):
    _, m_total, d = partial.shape
    m_half = m_total // 2
    rows = m_half // NC

    def body(p_ref, g_ref, out_ref, send_buf, recv_buf, send_sems, recv_sems):
        my_x = lax.axis_index("x")
        my_y = lax.axis_index("y")
        my_z = lax.axis_index("z")
        peer = (1 - my_x, my_y, my_z)
        peer_base = (1 - my_x) * m_half
        my_base = my_x * m_half

        barrier_sem = pltpu.get_barrier_semaphore()
        pl.semaphore_signal(
            barrier_sem,
            inc=1,
            device_id=peer,
            device_id_type=pl.DeviceIdType.MESH,
        )
        send_buf[0] = p_ref[0, pl.ds(peer_base, rows), :].astype(jnp.bfloat16)
        pl.semaphore_wait(barrier_sem, 1)

        rdmas = []
        for c in range(NC):
            rdma = pltpu.make_async_remote_copy(
                src_ref=send_buf.at[c],
                dst_ref=recv_buf.at[c],
                send_sem=send_sems.at[c],
                recv_sem=recv_sems.at[c],
                device_id=peer,
                device_id_type=pl.DeviceIdType.MESH,
            )
            rdma.start()
            rdmas.append(rdma)
            if c + 1 < NC:
                send_buf[c + 1] = p_ref[
                    0, pl.ds(peer_base + (c + 1) * rows, rows), :
                ].astype(jnp.bfloat16)

        for c in range(NC):
            rdmas[c].wait_recv()
            y = p_ref[0, pl.ds(my_base + c * rows, rows), :] + recv_buf[c].astype(
                jnp.float32
            )
            rms = jnp.sqrt(jnp.mean(y * y, axis=-1, keepdims=True) + 1e-6)
            out_ref[pl.ds(c * rows, rows), :] = y / rms * g_ref[...]

        for c in range(NC):
            rdmas[c].wait_send()

    gamma2 = gamma.reshape(1, d)
    return pl.pallas_call(
        body,
        out_shape=jax.ShapeDtypeStruct((m_half, d), jnp.float32),
        in_specs=[
            pl.BlockSpec(memory_space=pltpu.VMEM),
            pl.BlockSpec(memory_space=pltpu.VMEM),
        ],
        out_specs=pl.BlockSpec(memory_space=pltpu.VMEM),
        scratch_shapes=[
            pltpu.VMEM((NC, rows, d), jnp.bfloat16),
            pltpu.VMEM((NC, rows, d), jnp.bfloat16),
            pltpu.SemaphoreType.DMA((NC,)),
            pltpu.SemaphoreType.DMA((NC,)),
        ],
        compiler_params=pltpu.CompilerParams(collective_id=0),
    )(partial, gamma2)


# device time: 11557 ns/iter; 1.1403x vs baseline; 1.1278x over previous
import jax
import jax.numpy as jnp
from jax import lax
from jax.experimental import pallas as pl
from jax.experimental.pallas import tpu as pltpu

NS = 4


def kernel(partial, gamma):
    _, m_total, d = partial.shape
    m_half = m_total // 2
    q_rows = m_half // 4
    rows = q_rows // NS

    def body(
        p_ref,
        g_ref,
        out_ref,
        send_x,
        recv_x,
        outq,
        recv_y,
        recv_z,
        sx_sems,
        rx_sems,
        sy_sems,
        ry_sems,
        sz_sems,
        rz_sems,
    ):
        my_x = lax.axis_index("x")
        my_y = lax.axis_index("y")
        my_z = lax.axis_index("z")
        xpeer = (1 - my_x, my_y, my_z)
        ynbr = (my_x, 1 - my_y, my_z)
        znbr = (my_x, my_y, 1 - my_z)

        qi = 2 * my_y + my_z
        qi_y = 2 * (1 - my_y) + my_z
        qi_z = 2 * my_y + (1 - my_z)
        qi_d = 2 * (1 - my_y) + (1 - my_z)

        my_base = my_x * m_half
        peer_base = (1 - my_x) * m_half

        barrier_sem = pltpu.get_barrier_semaphore()
        for nbr in (xpeer, ynbr, znbr):
            pl.semaphore_signal(
                barrier_sem,
                inc=1,
                device_id=nbr,
                device_id_type=pl.DeviceIdType.MESH,
            )

        def stage(c):
            src_q = qi if c < NS else qi_d
            s = c % NS
            send_x[c] = p_ref[
                0, pl.ds(peer_base + src_q * q_rows + s * rows, rows), :
            ].astype(jnp.bfloat16)

        stage(0)
        pl.semaphore_wait(barrier_sem, 3)

        x_rdmas = []
        for c in range(2 * NS):
            rdma = pltpu.make_async_remote_copy(
                src_ref=send_x.at[c],
                dst_ref=recv_x.at[c],
                send_sem=sx_sems.at[c],
                recv_sem=rx_sems.at[c],
                device_id=xpeer,
                device_id_type=pl.DeviceIdType.MESH,
            )
            rdma.start()
            x_rdmas.append(rdma)
            if c + 1 < 2 * NS:
                stage(c + 1)

        g = g_ref[...]

        def norm(base_row, s, contrib):
            y = p_ref[0, pl.ds(my_base + base_row + s * rows, rows), :] + contrib
            inv = lax.rsqrt(jnp.mean(y * y, axis=-1, keepdims=True) + 1e-6)
            return y * inv * g

        face_rdmas = []
        for s in range(NS):
            x_rdmas[s].wait_recv()
            o = norm(qi * q_rows, s, recv_x[s].astype(jnp.float32))
            out_ref[pl.ds(qi * q_rows + s * rows, rows), :] = o
            outq[s] = o.astype(jnp.bfloat16)
            for dst, dst_buf, ssems, rsems in (
                (ynbr, recv_y, sy_sems, ry_sems),
                (znbr, recv_z, sz_sems, rz_sems),
            ):
                rdma = pltpu.make_async_remote_copy(
                    src_ref=outq.at[s],
                    dst_ref=dst_buf.at[s],
                    send_sem=ssems.at[s],
                    recv_sem=rsems.at[s],
                    device_id=dst,
                    device_id_type=pl.DeviceIdType.MESH,
                )
                rdma.start()
                face_rdmas.append(rdma)

        for s in range(NS):
            x_rdmas[NS + s].wait_recv()
            o = norm(qi_d * q_rows, s, recv_x[NS + s].astype(jnp.float32))
            out_ref[pl.ds(qi_d * q_rows + s * rows, rows), :] = o

        for src_qi, buf, rsems in ((qi_y, recv_y, ry_sems), (qi_z, recv_z, rz_sems)):
            for s in range(NS):
                pltpu.make_async_remote_copy(
                    src_ref=outq.at[s],
                    dst_ref=buf.at[s],
                    send_sem=rsems.at[s],
                    recv_sem=rsems.at[s],
                    device_id=xpeer,
                    device_id_type=pl.DeviceIdType.MESH,
                ).wait_recv()
                out_ref[pl.ds(src_qi * q_rows + s * rows, rows), :] = buf[s].astype(
                    jnp.float32
                )

        for rdma in x_rdmas + face_rdmas:
            rdma.wait_send()

    gamma2 = gamma[None, :]
    return pl.pallas_call(
        body,
        out_shape=jax.ShapeDtypeStruct((m_half, d), jnp.float32),
        in_specs=[
            pl.BlockSpec(memory_space=pltpu.VMEM),
            pl.BlockSpec(memory_space=pltpu.VMEM),
        ],
        out_specs=pl.BlockSpec(memory_space=pltpu.VMEM),
        scratch_shapes=[
            pltpu.VMEM((2 * NS, rows, d), jnp.bfloat16),
            pltpu.VMEM((2 * NS, rows, d), jnp.bfloat16),
            pltpu.VMEM((NS, rows, d), jnp.bfloat16),
            pltpu.VMEM((NS, rows, d), jnp.bfloat16),
            pltpu.VMEM((NS, rows, d), jnp.bfloat16),
            pltpu.SemaphoreType.DMA((2 * NS,)),
            pltpu.SemaphoreType.DMA((2 * NS,)),
            pltpu.SemaphoreType.DMA((NS,)),
            pltpu.SemaphoreType.DMA((NS,)),
            pltpu.SemaphoreType.DMA((NS,)),
            pltpu.SemaphoreType.DMA((NS,)),
        ],
        compiler_params=pltpu.CompilerParams(collective_id=0),
    )(partial, gamma2)
